# baseline (speedup 1.0000x reference)
import functools
import math

import jax
import jax.numpy as jnp
from jax import lax
from jax.experimental import pallas as pl
from jax.experimental.pallas import tpu as pltpu
from jax.experimental.pallas import tpu_sc as plsc

D_MODEL = 64
SCALE = math.sqrt(D_MODEL)

NUM_CORES = 2
NUM_SUBCORES = 16
NUM_WORKERS = NUM_CORES * NUM_SUBCORES
LANES = 16

CHUNK = 256
GATHER = 128


def _emb_body(n_chunks, seq_len, batch, x_hbm, table_hbm, out_hbm,
              idx_v, rows_v, trows_v, isem0, isem1, gsem0, gsem1,
              osem0, osem1):
    isems = (isem0, isem1)
    gsems = (gsem0, gsem1)
    osems = (osem0, osem1)

    wid = lax.axis_index("s") * NUM_CORES + lax.axis_index("c")
    b0 = wid * (2 * CHUNK)
    n = n_chunks

    def boff(g):
        return (g >> 1), b0 + (g & 1) * CHUNK

    def fire_idx(g, q):
        t, b = boff(g)
        pltpu.async_copy(x_hbm.at[t, pl.ds(b, CHUNK)], idx_v.at[q], isems[q])

    def wait_idx(q):
        pltpu.make_async_copy(x_hbm.at[0, pl.ds(0, CHUNK)], idx_v.at[q],
                              isems[q]).wait()

    def fire_gathers(g, a):
        for j in range(CHUNK // GATHER):
            pltpu.async_copy(
                table_hbm.at[idx_v.at[a, pl.ds(j * GATHER, GATHER)]],
                rows_v.at[a, pl.ds(j * GATHER, GATHER)],
                gsems[a],
            )

    def wait_gathers(a):
        pltpu.make_async_copy(table_hbm.at[pl.ds(0, CHUNK)],
                              rows_v.at[a], gsems[a]).wait()

    def fire_writeout(g, a):
        t, b = boff(g)
        pltpu.async_copy(trows_v.at[a], out_hbm.at[t, :, pl.ds(b, CHUNK)],
                         osems[a])

    def wait_writeout(a):
        pltpu.make_async_copy(trows_v.at[a], out_hbm.at[0, :, pl.ds(0, CHUNK)],
                              osems[a]).wait()

    base_iota = lax.iota(jnp.int32, LANES)

    def transpose_scale(a):
        rows = rows_v.at[a]

        @plsc.parallel_loop(0, D_MODEL, 1, unroll=2)
        def _(d):
            cols = jnp.full((LANES,), d, jnp.int32)
            for j in range(CHUNK // LANES):
                rids = base_iota + (j * LANES)
                val = plsc.load_gather(rows, [rids, cols])
                trows_v[a, d, pl.ds(j * LANES, LANES)] = val * SCALE

    fire_idx(0, 0)
    fire_idx(1, 1)
    wait_idx(0)
    fire_gathers(0, 0)

    def outer(i, carry):
        for a in range(2):
            g = i * 2 + a

            @pl.when(g + 1 < n)
            def _():
                wait_idx(1 - a)
                fire_gathers(g + 1, 1 - a)

            wait_gathers(a)

            @pl.when(g + 2 < n)
            def _():
                fire_idx(g + 2, a)

            @pl.when(g >= 2)
            def _():
                wait_writeout(a)

            transpose_scale(a)
            fire_writeout(g, a)
        return carry

    lax.fori_loop(0, n // 2, outer, 0)

    wait_writeout((n - 1) % 2)
    wait_writeout(n % 2)


def kernel(x, table):
    n_rows, n_cols = x.shape
    xt = jnp.transpose(x).astype(jnp.int32)

    stripe = 2 * CHUNK
    assert n_rows % (NUM_WORKERS * stripe) == 0 or \
        n_rows == NUM_WORKERS * stripe
    n_chunks = 2 * n_cols

    mesh = plsc.VectorSubcoreMesh(core_axis_name="c", subcore_axis_name="s")
    emb = pl.kernel(
        functools.partial(_emb_body, n_chunks, n_cols, n_rows),
        out_type=jax.ShapeDtypeStruct((n_cols, D_MODEL, n_rows), jnp.float32),
        mesh=mesh,
        scratch_types=(
            [pltpu.VMEM((2, CHUNK), jnp.int32),
             pltpu.VMEM((2, CHUNK, D_MODEL), jnp.float32),
             pltpu.VMEM((2, D_MODEL, CHUNK), jnp.float32)]
            + [pltpu.SemaphoreType.DMA] * 6
        ),
        compiler_params=pltpu.CompilerParams(use_tc_tiling_on_sc=False,
                                             needs_layout_passes=False),
    )
    out = emb(xt, table)
    return jnp.transpose(out, (2, 0, 1))


# speedup vs baseline: 1.9772x; 1.9772x over previous
import functools
import math

import jax
import jax.numpy as jnp
from jax import lax
from jax.experimental import pallas as pl
from jax.experimental.pallas import tpu as pltpu
from jax.experimental.pallas import tpu_sc as plsc

D_MODEL = 64
SCALE = math.sqrt(D_MODEL)

NUM_CORES = 2
NUM_SUBCORES = 16
NUM_WORKERS = NUM_CORES * NUM_SUBCORES
LANES = 16

CHUNK = 256
GATHER = 128


def _emb_body(n_chunks, seq_len, batch, x_hbm, table_hbm, out_hbm,
              idx_v, rows_v, trows_v, isem0, isem1, gsem0, gsem1,
              osem0, osem1):
    isems = (isem0, isem1)
    gsems = (gsem0, gsem1)
    osems = (osem0, osem1)

    wid = lax.axis_index("s") * NUM_CORES + lax.axis_index("c")
    b0 = wid * (2 * CHUNK)
    n = n_chunks

    def boff(g):
        return (g >> 1), b0 + (g & 1) * CHUNK

    def fire_idx(g, q):
        t, b = boff(g)
        pltpu.async_copy(x_hbm.at[t, pl.ds(b, CHUNK)], idx_v.at[q], isems[q])

    def wait_idx(q):
        pltpu.make_async_copy(x_hbm.at[0, pl.ds(0, CHUNK)], idx_v.at[q],
                              isems[q]).wait()

    def fire_gathers(g, a):
        for j in range(CHUNK // GATHER):
            pltpu.async_copy(
                table_hbm.at[idx_v.at[a, pl.ds(j * GATHER, GATHER)]],
                rows_v.at[a, pl.ds(j * GATHER, GATHER)],
                gsems[a],
            )

    def wait_gathers(a):
        pltpu.make_async_copy(table_hbm.at[pl.ds(0, CHUNK)],
                              rows_v.at[a], gsems[a]).wait()

    def fire_writeout(g, a):
        t, b = boff(g)
        pltpu.async_copy(trows_v.at[a, :, pl.ds(0, CHUNK)],
                         out_hbm.at[t, :, pl.ds(b, CHUNK)], osems[a])

    def wait_writeout(a):
        pltpu.make_async_copy(trows_v.at[a, :, pl.ds(0, CHUNK)],
                              out_hbm.at[0, :, pl.ds(0, CHUNK)],
                              osems[a]).wait()

    base_iota = lax.iota(jnp.int32, LANES)

    def transpose_scale(a):
        trows = trows_v.at[a]

        @plsc.parallel_loop(0, CHUNK, 1, unroll=4)
        def _(r):
            colv = jnp.full((LANES,), r, jnp.int32)
            for c in range(D_MODEL // LANES):
                d_ids = base_iota + (c * LANES)
                val = rows_v[a, r, pl.ds(c * LANES, LANES)] * SCALE
                plsc.store_scatter(trows, [d_ids, colv], val)

    fire_idx(0, 0)
    fire_idx(1, 1)
    wait_idx(0)
    fire_gathers(0, 0)

    def outer(i, carry):
        for a in range(2):
            g = i * 2 + a

            @pl.when(g + 1 < n)
            def _():
                wait_idx(1 - a)
                fire_gathers(g + 1, 1 - a)

            wait_gathers(a)

            @pl.when(g + 2 < n)
            def _():
                fire_idx(g + 2, a)

            @pl.when(g >= 2)
            def _():
                wait_writeout(a)

            transpose_scale(a)
            fire_writeout(g, a)
        return carry

    lax.fori_loop(0, n // 2, outer, 0)

    wait_writeout((n - 1) % 2)
    wait_writeout(n % 2)


def kernel(x, table):
    n_rows, n_cols = x.shape
    xt = jnp.transpose(x).astype(jnp.int32)

    stripe = 2 * CHUNK
    assert n_rows % (NUM_WORKERS * stripe) == 0 or \
        n_rows == NUM_WORKERS * stripe
    n_chunks = 2 * n_cols

    mesh = plsc.VectorSubcoreMesh(core_axis_name="c", subcore_axis_name="s")
    emb = pl.kernel(
        functools.partial(_emb_body, n_chunks, n_cols, n_rows),
        out_type=jax.ShapeDtypeStruct((n_cols, D_MODEL, n_rows), jnp.float32),
        mesh=mesh,
        scratch_types=(
            [pltpu.VMEM((2, CHUNK), jnp.int32),
             pltpu.VMEM((2, CHUNK, D_MODEL), jnp.float32),
             pltpu.VMEM((2, D_MODEL, CHUNK + 1), jnp.float32)]
            + [pltpu.SemaphoreType.DMA] * 6
        ),
        compiler_params=pltpu.CompilerParams(use_tc_tiling_on_sc=False,
                                             needs_layout_passes=False),
    )
    out = emb(xt, table)
    return jnp.transpose(out, (2, 0, 1))


# speedup vs baseline: 3.3654x; 1.7021x over previous
import functools
import math

import jax
import jax.numpy as jnp
from jax import lax
from jax.experimental import pallas as pl
from jax.experimental.pallas import tpu as pltpu
from jax.experimental.pallas import tpu_sc as plsc

D_MODEL = 64
SCALE = math.sqrt(D_MODEL)

NUM_CORES = 2
NUM_SUBCORES = 16
NUM_WORKERS = NUM_CORES * NUM_SUBCORES
LANES = 16

CHUNK = 256
GATHER = 128


def _emb_body(n_chunks, seq_len, batch, x_hbm, table_hbm, out_hbm,
              idx_v, rows_v, trows_v, isem0, isem1, gsem0, gsem1,
              osem0, osem1):
    isems = (isem0, isem1)
    gsems = (gsem0, gsem1)
    osems = (osem0, osem1)

    wid = lax.axis_index("s") * NUM_CORES + lax.axis_index("c")
    b0 = wid * (2 * CHUNK)
    n = n_chunks

    def boff(g):
        return (g >> 1), b0 + (g & 1) * CHUNK

    def fire_idx(g, q):
        t, b = boff(g)
        pltpu.async_copy(x_hbm.at[t, pl.ds(b, CHUNK)], idx_v.at[q], isems[q])

    def wait_idx(q):
        pltpu.make_async_copy(x_hbm.at[0, pl.ds(0, CHUNK)], idx_v.at[q],
                              isems[q]).wait()

    def fire_gathers(g, a):
        for j in range(CHUNK // GATHER):
            pltpu.async_copy(
                table_hbm.at[idx_v.at[a, pl.ds(j * GATHER, GATHER)]],
                rows_v.at[a, pl.ds(j * GATHER, GATHER)],
                gsems[a],
            )

    def wait_gathers(a):
        pltpu.make_async_copy(table_hbm.at[pl.ds(0, CHUNK)],
                              rows_v.at[a], gsems[a]).wait()

    def bcol0(g):
        return wid * 4 + (g & 1) * 2

    def fire_writeout(g, a):
        t = g >> 1
        bc = bcol0(g)
        for j in range(2):
            pltpu.async_copy(
                trows_v.at[a, :, :, pl.ds(j * 128, 128)],
                out_hbm.at[t, :, bc + j, :, :], osems[a])

    def wait_writeout(a):
        for j in range(2):
            pltpu.make_async_copy(trows_v.at[a, :, :, pl.ds(j * 128, 128)],
                                  out_hbm.at[0, :, 0, :, :],
                                  osems[a]).wait()

    base_iota = lax.iota(jnp.int32, LANES)
    dsplit = [((base_iota + c * LANES) // 8, (base_iota + c * LANES) % 8)
              for c in range(D_MODEL // LANES)]

    def transpose_scale(a):
        trows = trows_v.at[a]

        @plsc.parallel_loop(0, CHUNK, 1, unroll=4)
        def _(r):
            colv = jnp.full((LANES,), r, jnp.int32)
            for c in range(D_MODEL // LANES):
                dr_ids, drow_ids = dsplit[c]
                val = rows_v[a, r, pl.ds(c * LANES, LANES)] * SCALE
                plsc.store_scatter(trows, [dr_ids, drow_ids, colv], val)

    fire_idx(0, 0)
    fire_idx(1, 1)
    wait_idx(0)
    fire_gathers(0, 0)

    def outer(i, carry):
        for a in range(2):
            g = i * 2 + a

            @pl.when(g + 1 < n)
            def _():
                wait_idx(1 - a)
                fire_gathers(g + 1, 1 - a)

            wait_gathers(a)

            @pl.when(g + 2 < n)
            def _():
                fire_idx(g + 2, a)

            @pl.when(g >= 2)
            def _():
                wait_writeout(a)

            transpose_scale(a)
            fire_writeout(g, a)
        return carry

    lax.fori_loop(0, n // 2, outer, 0)

    wait_writeout((n - 1) % 2)
    wait_writeout(n % 2)


def kernel(x, table):
    n_rows, n_cols = x.shape
    xt = jnp.transpose(x).astype(jnp.int32)

    stripe = 2 * CHUNK
    assert n_rows % (NUM_WORKERS * stripe) == 0 or \
        n_rows == NUM_WORKERS * stripe
    n_chunks = 2 * n_cols

    mesh = plsc.VectorSubcoreMesh(core_axis_name="c", subcore_axis_name="s")
    emb = pl.kernel(
        functools.partial(_emb_body, n_chunks, n_cols, n_rows),
        out_type=jax.ShapeDtypeStruct(
            (n_cols, D_MODEL // 8, n_rows // 128, 8, 128), jnp.float32),
        mesh=mesh,
        scratch_types=(
            [pltpu.VMEM((2, CHUNK), jnp.int32),
             pltpu.VMEM((2, CHUNK, D_MODEL), jnp.float32),
             pltpu.VMEM((2, D_MODEL // 8, 8, CHUNK + 1), jnp.float32)]
            + [pltpu.SemaphoreType.DMA] * 6
        ),
        compiler_params=pltpu.CompilerParams(use_tc_tiling_on_sc=False,
                                             needs_layout_passes=False),
    )
    out5 = emb(xt, table)
    out = jnp.transpose(out5, (2, 4, 0, 1, 3))
    return out.reshape(n_rows, n_cols, D_MODEL)


# speedup vs baseline: 3.4573x; 1.0273x over previous
import functools
import math

import jax
import jax.numpy as jnp
from jax import lax
from jax.experimental import pallas as pl
from jax.experimental.pallas import tpu as pltpu
from jax.experimental.pallas import tpu_sc as plsc

D_MODEL = 64
SCALE = math.sqrt(D_MODEL)

NUM_CORES = 2
NUM_SUBCORES = 16
NUM_WORKERS = NUM_CORES * NUM_SUBCORES
LANES = 16

CHUNK = 256
GATHER = 128


def _emb_body(n_chunks, seq_len, batch, x_hbm, table_hbm, out_hbm,
              idx_v, rows_v, trows_v, *sems):
    isems = sems[0:4]
    gsems = sems[4:8]
    osems = sems[8:10]

    wid = lax.axis_index("s") * NUM_CORES + lax.axis_index("c")
    b0 = wid * (2 * CHUNK)
    n = n_chunks

    def boff(g):
        return (g >> 1), b0 + (g & 1) * CHUNK

    def fire_idx(g, q):
        t, b = boff(g)
        pltpu.async_copy(x_hbm.at[t, pl.ds(b, CHUNK)], idx_v.at[q], isems[q])

    def wait_idx(q):
        pltpu.make_async_copy(x_hbm.at[0, pl.ds(0, CHUNK)], idx_v.at[q],
                              isems[q]).wait()

    def fire_gathers(g, a):
        for j in range(CHUNK // GATHER):
            pltpu.async_copy(
                table_hbm.at[idx_v.at[a, pl.ds(j * GATHER, GATHER)]],
                rows_v.at[a, pl.ds(j * GATHER, GATHER)],
                gsems[a],
            )

    def wait_gathers(a):
        pltpu.make_async_copy(table_hbm.at[pl.ds(0, CHUNK)],
                              rows_v.at[a], gsems[a]).wait()

    def bcol0(g):
        return wid * 4 + (g & 1) * 2

    def fire_writeout(g, a):
        t = g >> 1
        bc = bcol0(g)
        for j in range(2):
            pltpu.async_copy(
                trows_v.at[a, :, :, pl.ds(j * 128, 128)],
                out_hbm.at[t, :, bc + j, :, :], osems[a])

    def wait_writeout(a):
        for j in range(2):
            pltpu.make_async_copy(trows_v.at[a, :, :, pl.ds(j * 128, 128)],
                                  out_hbm.at[0, :, 0, :, :],
                                  osems[a]).wait()

    base_iota = lax.iota(jnp.int32, LANES)
    dsplit = [((base_iota + c * LANES) // 8, (base_iota + c * LANES) % 8)
              for c in range(D_MODEL // LANES)]

    def transpose_scale(a4, a2):
        trows = trows_v.at[a2]

        @plsc.parallel_loop(0, CHUNK, 1, unroll=4)
        def _(r):
            colv = jnp.full((LANES,), r, jnp.int32)
            for c in range(D_MODEL // LANES):
                dr_ids, drow_ids = dsplit[c]
                val = rows_v[a4, r, pl.ds(c * LANES, LANES)] * SCALE
                plsc.store_scatter(trows, [dr_ids, drow_ids, colv], val)

    for q in range(4):
        fire_idx(q, q)
    wait_idx(0)
    fire_gathers(0, 0)
    wait_idx(1)
    fire_gathers(1, 1)

    def outer(i, carry):
        for a4 in range(4):
            g = i * 4 + a4
            a2 = a4 % 2

            @pl.when(g + 2 < n)
            def _():
                wait_idx((a4 + 2) % 4)
                fire_gathers(g + 2, (a4 + 2) % 4)

            wait_gathers(a4)

            @pl.when(g + 4 < n)
            def _():
                fire_idx(g + 4, a4)

            @pl.when(g >= 2)
            def _():
                wait_writeout(a2)

            transpose_scale(a4, a2)
            fire_writeout(g, a2)
        return carry

    lax.fori_loop(0, n // 4, outer, 0)

    wait_writeout((n - 1) % 2)
    wait_writeout(n % 2)


def kernel(x, table):
    n_rows, n_cols = x.shape
    xt = jnp.transpose(x).astype(jnp.int32)

    stripe = 2 * CHUNK
    assert n_rows % (NUM_WORKERS * stripe) == 0 or \
        n_rows == NUM_WORKERS * stripe
    n_chunks = 2 * n_cols

    mesh = plsc.VectorSubcoreMesh(core_axis_name="c", subcore_axis_name="s")
    emb = pl.kernel(
        functools.partial(_emb_body, n_chunks, n_cols, n_rows),
        out_type=jax.ShapeDtypeStruct(
            (n_cols, D_MODEL // 8, n_rows // 128, 8, 128), jnp.float32),
        mesh=mesh,
        scratch_types=(
            [pltpu.VMEM((4, CHUNK), jnp.int32),
             pltpu.VMEM((4, CHUNK, D_MODEL), jnp.float32),
             pltpu.VMEM((2, D_MODEL // 8, 8, CHUNK + 1), jnp.float32)]
            + [pltpu.SemaphoreType.DMA] * 10
        ),
        compiler_params=pltpu.CompilerParams(use_tc_tiling_on_sc=False,
                                             needs_layout_passes=False),
    )
    out5 = emb(xt, table)
    out = jnp.transpose(out5, (2, 4, 0, 1, 3))
    return out.reshape(n_rows, n_cols, D_MODEL)
